# Initial kernel scaffold; baseline (speedup 1.0000x reference)
import jax
import jax.numpy as jnp
from jax.experimental import pallas as pl


def kernel(edge_index_img, edge_weight_img, edge_index_txt, edge_weight_txt, image_preference, text_preference, image_repre, text_repre):
    raise NotImplementedError("write your pallas kernel here")


# speedup vs baseline: 2.4571x; 2.4571x over previous
import functools

import jax
import jax.numpy as jnp
from jax import lax
from jax.experimental import pallas as pl
from jax.experimental.pallas import tpu as pltpu
from jax.experimental.pallas import tpu_sc as plsc

N_USERS = 4000
N_ITEMS = 6000
N = N_USERS + N_ITEMS
E = 320000
D = 128
N_LAYERS = 2
DELTA = 0.5

NC = 2
NS = 16
CHUNK = 128
NCHUNK = 160
BLKC = 16
NBLK = NCHUNK // BLKC
EPW = NCHUNK * CHUNK
EPAD = EPW * NS
NP = 10240
ROWS_PER_TILE = NP // NS
ROW_CHUNK = 128
NROWCH = ROWS_PER_TILE // ROW_CHUNK
LANES = 16
DSUB = D // LANES


def _sc_body(src_hbm, dst_hbm, w_hbm, x0_hbm, x1_hbm, x2_hbm,
             src_v, dst_v, w_v, rows_v, acc_sh, sem):
    c = lax.axis_index("c")
    s = lax.axis_index("s")
    wid = c * NS + s

    def run_layer(x_in, x_out):
        gbase = c * NP + s * ROWS_PER_TILE
        lbase = s * ROWS_PER_TILE

        def init_chunk(r, _):
            pltpu.sync_copy(x_in.at[pl.ds(gbase + r * ROW_CHUNK, ROW_CHUNK)],
                            rows_v.at[pl.ds(0, ROW_CHUNK)])

            def scale_row(i, _):
                for d in range(DSUB):
                    v = rows_v[i, pl.ds(d * LANES, LANES)]
                    rows_v[i, pl.ds(d * LANES, LANES)] = v * DELTA
                return ()

            lax.fori_loop(0, ROW_CHUNK, scale_row, ())
            pltpu.sync_copy(rows_v.at[pl.ds(0, ROW_CHUNK)],
                            acc_sh.at[pl.ds(lbase + r * ROW_CHUNK, ROW_CHUNK)])
            return ()

        lax.fori_loop(0, NROWCH, init_chunk, ())
        plsc.subcore_barrier()

        def edge_block(b, _):
            pltpu.sync_copy(src_hbm.at[wid, pl.ds(b * BLKC, BLKC)], src_v)
            pltpu.sync_copy(dst_hbm.at[wid, pl.ds(b * BLKC, BLKC)], dst_v)
            pltpu.sync_copy(w_hbm.at[wid, pl.ds(b * BLKC, BLKC)], w_v)

            def edge_chunk(j, _):
                pltpu.async_copy(x_in.at[src_v.at[j]],
                                 rows_v.at[pl.ds(0, CHUNK)], sem).wait()

                def scale_group(g, _):
                    wvec = w_v[j, pl.ds(g * LANES, LANES)]
                    for l in range(LANES):
                        w = wvec[l]
                        e = g * LANES + l
                        for d in range(DSUB):
                            v = rows_v[e, pl.ds(d * LANES, LANES)]
                            rows_v[e, pl.ds(d * LANES, LANES)] = v * w
                    return ()

                lax.fori_loop(0, CHUNK // LANES, scale_group, ())
                pltpu.sync_copy(rows_v.at[pl.ds(0, CHUNK)],
                                acc_sh.at[dst_v.at[j]], add=True)
                return ()

            lax.fori_loop(0, BLKC, edge_chunk, ())
            return ()

        lax.fori_loop(0, NBLK, edge_block, ())
        plsc.subcore_barrier()

        def wb_chunk(r, _):
            pltpu.sync_copy(acc_sh.at[pl.ds(lbase + r * ROW_CHUNK, ROW_CHUNK)],
                            rows_v.at[pl.ds(0, ROW_CHUNK)])
            pltpu.sync_copy(rows_v.at[pl.ds(0, ROW_CHUNK)],
                            x_out.at[pl.ds(gbase + r * ROW_CHUNK, ROW_CHUNK)])
            return ()

        lax.fori_loop(0, NROWCH, wb_chunk, ())
        plsc.subcore_barrier()

    run_layer(x0_hbm, x1_hbm)
    run_layer(x1_hbm, x2_hbm)


@jax.jit
def _mce_sc(src_all, dst_all, w_all, x0):
    k = functools.partial(
        pl.kernel,
        out_type=[jax.ShapeDtypeStruct((NC * NP, D), jnp.float32),
                  jax.ShapeDtypeStruct((NC * NP, D), jnp.float32)],
        mesh=plsc.VectorSubcoreMesh(core_axis_name="c", subcore_axis_name="s"),
        scratch_types=[
            pltpu.VMEM((BLKC, CHUNK), jnp.int32),
            pltpu.VMEM((BLKC, CHUNK), jnp.int32),
            pltpu.VMEM((BLKC, CHUNK), jnp.float32),
            pltpu.VMEM((CHUNK, D), jnp.float32),
            pltpu.VMEM_SHARED((NP, D), jnp.float32),
            pltpu.SemaphoreType.DMA,
        ],
    )(_sc_body)
    x1, x2 = k(src_all, dst_all, w_all, x0)
    return x2


def _prep_edges(edge_index, edge_weight, src_offset):
    src = edge_index[0].astype(jnp.int32) + src_offset
    dst = edge_index[1].astype(jnp.int32)
    w = edge_weight[:, 0]
    pad = EPAD - E
    src = jnp.concatenate([src, jnp.zeros((pad,), jnp.int32)])
    dst = jnp.concatenate([dst, jnp.zeros((pad,), jnp.int32)])
    w = jnp.concatenate([w, jnp.zeros((pad,), jnp.float32)])
    return (src.reshape(NS, NCHUNK, CHUNK), dst.reshape(NS, NCHUNK, CHUNK),
            w.reshape(NS, NCHUNK, CHUNK))


def kernel(edge_index_img, edge_weight_img, edge_index_txt, edge_weight_txt,
           image_preference, text_preference, image_repre, text_repre):
    si, di, wi = _prep_edges(edge_index_img, edge_weight_img, 0)
    st, dt, wt = _prep_edges(edge_index_txt, edge_weight_txt, NP)
    src_all = jnp.concatenate([si, st], axis=0)
    dst_all = jnp.concatenate([di, dt], axis=0)
    w_all = jnp.concatenate([wi, wt], axis=0)
    zpad = jnp.zeros((NP - N, D), jnp.float32)
    x0 = jnp.concatenate([image_preference, image_repre, zpad,
                          text_preference, text_repre, zpad], axis=0)
    x2 = _mce_sc(src_all, dst_all, w_all, x0)
    user_preference = jnp.concatenate([x2[:N_USERS], x2[NP:NP + N_USERS]],
                                      axis=1)
    items = jnp.concatenate([x2[N_USERS:N], x2[NP + N_USERS:NP + N]], axis=1)
    return (user_preference, items)


# speedup vs baseline: 2.7815x; 1.1320x over previous
import functools

import jax
import jax.numpy as jnp
from jax import lax
from jax.experimental import pallas as pl
from jax.experimental.pallas import tpu as pltpu
from jax.experimental.pallas import tpu_sc as plsc

N_USERS = 4000
N_ITEMS = 6000
N = N_USERS + N_ITEMS
E = 320000
D = 128
N_LAYERS = 2
DELTA = 0.5

NC = 2
NS = 16
CHUNK = 128
NCHUNK = 160
BLKC = 16
NBLK = NCHUNK // BLKC
EPW = NCHUNK * CHUNK
EPAD = EPW * NS
NP = 10240
ROWS_PER_TILE = NP // NS
ROW_CHUNK = 128
NROWCH = ROWS_PER_TILE // ROW_CHUNK
LANES = 16
DSUB = D // LANES


def _sc_body(src_hbm, dst_hbm, w_hbm, x0_hbm, x1_hbm, x2_hbm,
             src_v, dst_v, w_v, rows_v, rows_v2, acc_sh, sem, sem2):
    c = lax.axis_index("c")
    s = lax.axis_index("s")
    wid = c * NS + s

    def run_layer(x_in, x_out):
        gbase = c * NP + s * ROWS_PER_TILE
        lbase = s * ROWS_PER_TILE

        def init_chunk(r, _):
            pltpu.sync_copy(x_in.at[pl.ds(gbase + r * ROW_CHUNK, ROW_CHUNK)],
                            rows_v.at[pl.ds(0, ROW_CHUNK)])

            def scale_row(i, _):
                for d in range(DSUB):
                    v = rows_v[i, pl.ds(d * LANES, LANES)]
                    rows_v[i, pl.ds(d * LANES, LANES)] = v * DELTA
                return ()

            lax.fori_loop(0, ROW_CHUNK, scale_row, ())
            pltpu.sync_copy(rows_v.at[pl.ds(0, ROW_CHUNK)],
                            acc_sh.at[pl.ds(lbase + r * ROW_CHUNK, ROW_CHUNK)])
            return ()

        lax.fori_loop(0, NROWCH, init_chunk, ())
        plsc.subcore_barrier()

        def gather(j, buf, s):
            return pltpu.make_async_copy(x_in.at[src_v.at[j]], buf, s)

        def scale_scatter(j, buf):
            def scale_group(g, _):
                wvec = w_v[j, pl.ds(g * LANES, LANES)]
                for l in range(LANES):
                    w = wvec[l]
                    e = g * LANES + l
                    for d in range(DSUB):
                        v = buf[e, pl.ds(d * LANES, LANES)]
                        buf[e, pl.ds(d * LANES, LANES)] = v * w
                return ()

            lax.fori_loop(0, CHUNK // LANES, scale_group, ())
            pltpu.sync_copy(buf.at[pl.ds(0, CHUNK)], acc_sh.at[dst_v.at[j]],
                            add=True)

        def edge_block(b, _):
            pltpu.sync_copy(src_hbm.at[wid, pl.ds(b * BLKC, BLKC)], src_v)
            pltpu.sync_copy(dst_hbm.at[wid, pl.ds(b * BLKC, BLKC)], dst_v)
            pltpu.sync_copy(w_hbm.at[wid, pl.ds(b * BLKC, BLKC)], w_v)

            gather(0, rows_v, sem).start()

            def chunk_pair(jp, _):
                j0 = 2 * jp
                j1 = j0 + 1
                gather(j1, rows_v2, sem2).start()
                gather(j0, rows_v, sem).wait()
                scale_scatter(j0, rows_v)

                @pl.when(j1 + 1 < BLKC)
                def _():
                    gather(j1 + 1, rows_v, sem).start()

                gather(j1, rows_v2, sem2).wait()
                scale_scatter(j1, rows_v2)
                return ()

            lax.fori_loop(0, BLKC // 2, chunk_pair, ())
            return ()

        lax.fori_loop(0, NBLK, edge_block, ())
        plsc.subcore_barrier()

        def wb_chunk(r, _):
            pltpu.sync_copy(acc_sh.at[pl.ds(lbase + r * ROW_CHUNK, ROW_CHUNK)],
                            rows_v.at[pl.ds(0, ROW_CHUNK)])
            pltpu.sync_copy(rows_v.at[pl.ds(0, ROW_CHUNK)],
                            x_out.at[pl.ds(gbase + r * ROW_CHUNK, ROW_CHUNK)])
            return ()

        lax.fori_loop(0, NROWCH, wb_chunk, ())
        plsc.subcore_barrier()

    run_layer(x0_hbm, x1_hbm)
    run_layer(x1_hbm, x2_hbm)


@jax.jit
def _mce_sc(src_all, dst_all, w_all, x0):
    k = functools.partial(
        pl.kernel,
        out_type=[jax.ShapeDtypeStruct((NC * NP, D), jnp.float32),
                  jax.ShapeDtypeStruct((NC * NP, D), jnp.float32)],
        mesh=plsc.VectorSubcoreMesh(core_axis_name="c", subcore_axis_name="s"),
        scratch_types=[
            pltpu.VMEM((BLKC, CHUNK), jnp.int32),
            pltpu.VMEM((BLKC, CHUNK), jnp.int32),
            pltpu.VMEM((BLKC, CHUNK), jnp.float32),
            pltpu.VMEM((CHUNK, D), jnp.float32),
            pltpu.VMEM((CHUNK, D), jnp.float32),
            pltpu.VMEM_SHARED((NP, D), jnp.float32),
            pltpu.SemaphoreType.DMA,
            pltpu.SemaphoreType.DMA,
        ],
    )(_sc_body)
    x1, x2 = k(src_all, dst_all, w_all, x0)
    return x2


def _prep_edges(edge_index, edge_weight, src_offset):
    src = edge_index[0].astype(jnp.int32) + src_offset
    dst = edge_index[1].astype(jnp.int32)
    w = edge_weight[:, 0]
    pad = EPAD - E
    src = jnp.concatenate([src, jnp.zeros((pad,), jnp.int32)])
    dst = jnp.concatenate([dst, jnp.zeros((pad,), jnp.int32)])
    w = jnp.concatenate([w, jnp.zeros((pad,), jnp.float32)])
    return (src.reshape(NS, NCHUNK, CHUNK), dst.reshape(NS, NCHUNK, CHUNK),
            w.reshape(NS, NCHUNK, CHUNK))


def kernel(edge_index_img, edge_weight_img, edge_index_txt, edge_weight_txt,
           image_preference, text_preference, image_repre, text_repre):
    si, di, wi = _prep_edges(edge_index_img, edge_weight_img, 0)
    st, dt, wt = _prep_edges(edge_index_txt, edge_weight_txt, NP)
    src_all = jnp.concatenate([si, st], axis=0)
    dst_all = jnp.concatenate([di, dt], axis=0)
    w_all = jnp.concatenate([wi, wt], axis=0)
    zpad = jnp.zeros((NP - N, D), jnp.float32)
    x0 = jnp.concatenate([image_preference, image_repre, zpad,
                          text_preference, text_repre, zpad], axis=0)
    x2 = _mce_sc(src_all, dst_all, w_all, x0)
    user_preference = jnp.concatenate([x2[:N_USERS], x2[NP:NP + N_USERS]],
                                      axis=1)
    items = jnp.concatenate([x2[N_USERS:N], x2[NP + N_USERS:NP + N]], axis=1)
    return (user_preference, items)


# speedup vs baseline: 2.7834x; 1.0007x over previous
import functools

import jax
import jax.numpy as jnp
from jax import lax
from jax.experimental import pallas as pl
from jax.experimental.pallas import tpu as pltpu
from jax.experimental.pallas import tpu_sc as plsc

N_USERS = 4000
N_ITEMS = 6000
N = N_USERS + N_ITEMS
E = 320000
D = 128
N_LAYERS = 2
DELTA = 0.5

NC = 2
NS = 16
CHUNK = 128
NCHUNK = 160
BLKC = 16
NBLK = NCHUNK // BLKC
EPW = NCHUNK * CHUNK
EPAD = EPW * NS
NP = 10240
ROWS_PER_TILE = NP // NS
ROW_CHUNK = 128
NROWCH = ROWS_PER_TILE // ROW_CHUNK
LANES = 16
DSUB = D // LANES


def _sc_body(src_hbm, dst_hbm, w_hbm, x0_hbm, x1_hbm, x2_hbm,
             src_v, dst_v, w_v, rows_v, rows_v2, acc_sh, sem, sem2):
    c = lax.axis_index("c")
    s = lax.axis_index("s")
    wid = c * NS + s

    def run_layer(x_in, x_out):
        gbase = c * NP + s * ROWS_PER_TILE
        lbase = s * ROWS_PER_TILE

        def init_chunk(r, _):
            pltpu.sync_copy(x_in.at[pl.ds(gbase + r * ROW_CHUNK, ROW_CHUNK)],
                            rows_v.at[pl.ds(0, ROW_CHUNK)])

            def scale_row(i, _):
                for d in range(DSUB):
                    v = rows_v[i, pl.ds(d * LANES, LANES)]
                    rows_v[i, pl.ds(d * LANES, LANES)] = v * DELTA
                return ()

            lax.fori_loop(0, ROW_CHUNK, scale_row, ())
            pltpu.sync_copy(rows_v.at[pl.ds(0, ROW_CHUNK)],
                            acc_sh.at[pl.ds(lbase + r * ROW_CHUNK, ROW_CHUNK)])
            return ()

        lax.fori_loop(0, NROWCH, init_chunk, ())
        plsc.subcore_barrier()

        def gather(j, buf, s):
            return pltpu.make_async_copy(x_in.at[src_v.at[j]], buf, s)

        def scatter_start(j, buf, s):
            pltpu.async_copy(buf.at[pl.ds(0, CHUNK)],
                             acc_sh.at[dst_v.at[j]], s, add=True)

        def scatter_wait(j, buf, s):
            pltpu.make_async_copy(buf.at[pl.ds(0, CHUNK)],
                                  acc_sh.at[dst_v.at[j]], s).wait()

        def scale(j, buf):
            def scale_group(g, _):
                wvec = w_v[j, pl.ds(g * LANES, LANES)]
                for l in range(LANES):
                    w = wvec[l]
                    e = g * LANES + l
                    for d in range(DSUB):
                        v = buf[e, pl.ds(d * LANES, LANES)]
                        buf[e, pl.ds(d * LANES, LANES)] = v * w
                return ()

            lax.fori_loop(0, CHUNK // LANES, scale_group, ())

        def edge_block(b, _):
            pltpu.sync_copy(src_hbm.at[wid, pl.ds(b * BLKC, BLKC)], src_v)
            pltpu.sync_copy(dst_hbm.at[wid, pl.ds(b * BLKC, BLKC)], dst_v)
            pltpu.sync_copy(w_hbm.at[wid, pl.ds(b * BLKC, BLKC)], w_v)

            gather(0, rows_v, sem).start()

            def step(j, buf, gsem, obuf, osem):
                gather(j, buf, gsem).wait()

                @pl.when(j > 0)
                def _():
                    scatter_wait(j - 1, obuf, osem)

                @pl.when(j + 1 < BLKC)
                def _():
                    gather(j + 1, obuf, osem).start()

                scale(j, buf)
                scatter_start(j, buf, gsem)

            def chunk_pair(jp, _):
                j0 = 2 * jp
                step(j0, rows_v, sem, rows_v2, sem2)
                step(j0 + 1, rows_v2, sem2, rows_v, sem)
                return ()

            lax.fori_loop(0, BLKC // 2, chunk_pair, ())
            scatter_wait(BLKC - 1, rows_v2, sem2)
            return ()

        lax.fori_loop(0, NBLK, edge_block, ())
        plsc.subcore_barrier()

        def wb_chunk(r, _):
            pltpu.sync_copy(acc_sh.at[pl.ds(lbase + r * ROW_CHUNK, ROW_CHUNK)],
                            rows_v.at[pl.ds(0, ROW_CHUNK)])
            pltpu.sync_copy(rows_v.at[pl.ds(0, ROW_CHUNK)],
                            x_out.at[pl.ds(gbase + r * ROW_CHUNK, ROW_CHUNK)])
            return ()

        lax.fori_loop(0, NROWCH, wb_chunk, ())
        plsc.subcore_barrier()

    run_layer(x0_hbm, x1_hbm)
    run_layer(x1_hbm, x2_hbm)


@jax.jit
def _mce_sc(src_all, dst_all, w_all, x0):
    k = functools.partial(
        pl.kernel,
        out_type=[jax.ShapeDtypeStruct((NC * NP, D), jnp.float32),
                  jax.ShapeDtypeStruct((NC * NP, D), jnp.float32)],
        mesh=plsc.VectorSubcoreMesh(core_axis_name="c", subcore_axis_name="s"),
        scratch_types=[
            pltpu.VMEM((BLKC, CHUNK), jnp.int32),
            pltpu.VMEM((BLKC, CHUNK), jnp.int32),
            pltpu.VMEM((BLKC, CHUNK), jnp.float32),
            pltpu.VMEM((CHUNK, D), jnp.float32),
            pltpu.VMEM((CHUNK, D), jnp.float32),
            pltpu.VMEM_SHARED((NP, D), jnp.float32),
            pltpu.SemaphoreType.DMA,
            pltpu.SemaphoreType.DMA,
        ],
    )(_sc_body)
    x1, x2 = k(src_all, dst_all, w_all, x0)
    return x2


def _prep_edges(edge_index, edge_weight, src_offset):
    src = edge_index[0].astype(jnp.int32) + src_offset
    dst = edge_index[1].astype(jnp.int32)
    w = edge_weight[:, 0]
    pad = EPAD - E
    src = jnp.concatenate([src, jnp.zeros((pad,), jnp.int32)])
    dst = jnp.concatenate([dst, jnp.zeros((pad,), jnp.int32)])
    w = jnp.concatenate([w, jnp.zeros((pad,), jnp.float32)])
    return (src.reshape(NS, NCHUNK, CHUNK), dst.reshape(NS, NCHUNK, CHUNK),
            w.reshape(NS, NCHUNK, CHUNK))


def kernel(edge_index_img, edge_weight_img, edge_index_txt, edge_weight_txt,
           image_preference, text_preference, image_repre, text_repre):
    si, di, wi = _prep_edges(edge_index_img, edge_weight_img, 0)
    st, dt, wt = _prep_edges(edge_index_txt, edge_weight_txt, NP)
    src_all = jnp.concatenate([si, st], axis=0)
    dst_all = jnp.concatenate([di, dt], axis=0)
    w_all = jnp.concatenate([wi, wt], axis=0)
    zpad = jnp.zeros((NP - N, D), jnp.float32)
    x0 = jnp.concatenate([image_preference, image_repre, zpad,
                          text_preference, text_repre, zpad], axis=0)
    x2 = _mce_sc(src_all, dst_all, w_all, x0)
    user_preference = jnp.concatenate([x2[:N_USERS], x2[NP:NP + N_USERS]],
                                      axis=1)
    items = jnp.concatenate([x2[N_USERS:N], x2[NP + N_USERS:NP + N]], axis=1)
    return (user_preference, items)
